# baseline (speedup 1.0000x reference)
import functools

import jax
import jax.numpy as jnp
from jax import lax
from jax.experimental import pallas as pl
from jax.experimental.pallas import tpu as pltpu
from jax.experimental.pallas import tpu_sc as plsc

NUM_CORES = 2
NUM_SUBCORES = 16
NLANES = 16
NW = NUM_CORES * NUM_SUBCORES


def _sc_histogram(xw, B, W, V):
    rows_per_w = B // NW
    chunk_rows = 64
    n_chunks = rows_per_w // chunk_rows
    n_groups = chunk_rows // NLANES

    mesh = plsc.VectorSubcoreMesh(
        core_axis_name="c", subcore_axis_name="s",
        num_cores=NUM_CORES, num_subcores=NUM_SUBCORES)

    @functools.partial(
        pl.kernel,
        out_type=jax.ShapeDtypeStruct((B, V), jnp.float32),
        mesh=mesh,
        compiler_params=pltpu.CompilerParams(
            needs_layout_passes=False, disable_bounds_checks=True,
            use_tc_tiling_on_sc=False),
        scratch_types=[
            pltpu.VMEM((chunk_rows, W), jnp.int32),
            pltpu.VMEM((chunk_rows, W), jnp.int32),
            pltpu.VMEM((rows_per_w, V), jnp.float32),
            pltpu.SemaphoreType.DMA,
            pltpu.SemaphoreType.DMA,
        ],
    )
    def hist(x_hbm, out_hbm, xb0, xb1, counts, sem0, sem1):
        wid = lax.axis_index("s") * NUM_CORES + lax.axis_index("c")
        row0 = wid * rows_per_w

        lane = lax.iota(jnp.int32, NLANES)
        ones = jnp.full((NLANES,), 1.0, jnp.float32)
        zeros = jnp.zeros((NLANES,), jnp.float32)
        mask6 = jnp.full((NLANES,), V - 1, jnp.int32)

        @plsc.parallel_loop(0, rows_per_w, unroll=8)
        def _(r):
            for k in range(V // NLANES):
                counts[r, pl.ds(k * NLANES, NLANES)] = zeros

        xbufs = (xb0, xb1)
        sems = (sem0, sem1)

        def start_chunk(c):
            return pltpu.async_copy(
                x_hbm.at[pl.ds(row0 + c * chunk_rows, chunk_rows), :],
                xbufs[c % 2], sems[c % 2])

        pending = start_chunk(0)
        for c in range(n_chunks):
            pending.wait()
            if c + 1 < n_chunks:
                pending = start_chunk(c + 1)
            xb = xbufs[c % 2]
            for g in range(n_groups):
                src_row = lane + g * NLANES
                dst_row = lane + (c * chunk_rows + g * NLANES)

                @plsc.parallel_loop(0, W, unroll=4)
                def _(w):
                    col = jnp.full((NLANES,), w, jnp.int32)
                    word = plsc.load_gather(xb, [src_row, col])
                    plsc.addupdate_scatter(
                        counts, [dst_row, word & mask6], ones)
                    plsc.addupdate_scatter(
                        counts,
                        [dst_row, (word >> 8) & mask6], ones)
                    plsc.addupdate_scatter(
                        counts,
                        [dst_row, (word >> 16) & mask6], ones)
                    plsc.addupdate_scatter(
                        counts, [dst_row, word >> 24], ones)

        pltpu.sync_copy(counts, out_hbm.at[pl.ds(row0, rows_per_w), :])

    return hist(xw)


def _tc_mlp(counts, emb, W1, b1, W2, b2, L):
    B, V = counts.shape
    E = emb.shape[1]
    H = W1.shape[1]
    P = W2.shape[1]
    blk = 2048
    inv_l = 1.0 / float(L)

    def body(c_ref, emb_ref, w1_ref, b1_ref, w2_ref, b2_ref, o_ref):
        ew = jnp.dot(emb_ref[...], w1_ref[...],
                     preferred_element_type=jnp.float32) * inv_l
        h = jnp.maximum(
            jnp.dot(c_ref[...], ew, preferred_element_type=jnp.float32)
            + b1_ref[...], 0.0)
        o = jnp.dot(h, w2_ref[...],
                    preferred_element_type=jnp.float32) + b2_ref[...]
        ss = jnp.sum(o * o, axis=1, keepdims=True)
        o_ref[...] = o / jnp.maximum(jnp.sqrt(ss), 1e-12)

    return pl.pallas_call(
        body,
        grid=(B // blk,),
        in_specs=[
            pl.BlockSpec((blk, V), lambda i: (i, 0)),
            pl.BlockSpec((V, E), lambda i: (0, 0)),
            pl.BlockSpec((E, H), lambda i: (0, 0)),
            pl.BlockSpec((1, H), lambda i: (0, 0)),
            pl.BlockSpec((H, P), lambda i: (0, 0)),
            pl.BlockSpec((1, P), lambda i: (0, 0)),
        ],
        out_specs=pl.BlockSpec((blk, P), lambda i: (i, 0)),
        out_shape=jax.ShapeDtypeStruct((B, P), jnp.float32),
    )(counts, emb, W1, b1.reshape(1, H), W2, b2.reshape(1, P))


def kernel(x, emb, W1, b1, W2, b2):
    B, L = x.shape
    V = emb.shape[0]
    assert B % (NW * NLANES) == 0 and L % 4 == 0
    xw = lax.bitcast_convert_type(
        x.astype(jnp.int8).reshape(B, L // 4, 4), jnp.int32)
    counts = _sc_histogram(xw, B, L // 4, V)
    return _tc_mlp(counts, emb, W1, b1, W2, b2, L)


# speedup vs baseline: 1.2253x; 1.2253x over previous
import functools

import jax
import jax.numpy as jnp
from jax import lax
from jax.experimental import pallas as pl
from jax.experimental.pallas import tpu as pltpu
from jax.experimental.pallas import tpu_sc as plsc

NUM_CORES = 2
NUM_SUBCORES = 16
NLANES = 16
NW = NUM_CORES * NUM_SUBCORES


def _sc_histogram(x_flat, B, L, V):
    rows_per_w = B // NW
    chunk_rows = 64
    n_chunks = rows_per_w // chunk_rows
    n_groups = chunk_rows // NLANES

    mesh = plsc.VectorSubcoreMesh(
        core_axis_name="c", subcore_axis_name="s",
        num_cores=NUM_CORES, num_subcores=NUM_SUBCORES)

    @functools.partial(
        pl.kernel,
        out_type=jax.ShapeDtypeStruct((B // 2, 2 * V), jnp.float32),
        mesh=mesh,
        compiler_params=pltpu.CompilerParams(
            needs_layout_passes=False, disable_bounds_checks=True,
            use_tc_tiling_on_sc=False),
        scratch_types=[
            pltpu.VMEM((chunk_rows * L,), jnp.int32),
            pltpu.VMEM((chunk_rows * L,), jnp.int32),
            pltpu.VMEM((rows_per_w // 2, 2 * V), jnp.float32),
            pltpu.SemaphoreType.DMA,
            pltpu.SemaphoreType.DMA,
        ],
    )
    def hist(x_hbm, out_hbm, xb0, xb1, counts, sem0, sem1):
        wid = lax.axis_index("s") * NUM_CORES + lax.axis_index("c")
        row0 = wid * rows_per_w

        lane = lax.iota(jnp.int32, NLANES)
        laneL = lane * L
        half = lane >> 1
        laneV = (lane & 1) * V
        ones = jnp.full((NLANES,), 1.0, jnp.float32)
        zeros = jnp.zeros((NLANES,), jnp.float32)

        @plsc.parallel_loop(0, rows_per_w // 2, unroll=8)
        def _(r):
            for k in range(2 * V // NLANES):
                counts[r, pl.ds(k * NLANES, NLANES)] = zeros

        xbufs = (xb0, xb1)
        sems = (sem0, sem1)

        def start_chunk(c):
            off = (row0 + c * chunk_rows) * L
            return pltpu.async_copy(
                x_hbm.at[pl.ds(off, chunk_rows * L)],
                xbufs[c % 2], sems[c % 2])

        pending = start_chunk(0)
        for c in range(n_chunks):
            pending.wait()
            if c + 1 < n_chunks:
                pending = start_chunk(c + 1)
            xb = xbufs[c % 2]
            for g in range(n_groups):
                src_base = laneL + (g * NLANES * L)
                dst_row = half + ((c * chunk_rows + g * NLANES) // 2)

                @plsc.parallel_loop(0, L, unroll=8)
                def _(l):
                    v = plsc.load_gather(xb, [src_base + l])
                    plsc.addupdate_scatter(
                        counts, [dst_row, laneV + v], ones)

        pltpu.sync_copy(
            counts, out_hbm.at[pl.ds(row0 // 2, rows_per_w // 2), :])

    return hist(x_flat)


def _tc_mlp(counts2, embd, w1d, b1d, w2d, b2d, L, P):
    B2, V2 = counts2.shape
    E2 = embd.shape[1]
    H2 = w1d.shape[1]
    blk = 1024
    inv_l = 1.0 / float(L)

    def body(c_ref, emb_ref, w1_ref, b1_ref, w2_ref, b2_ref, o_ref):
        ew = jnp.dot(emb_ref[...], w1_ref[...],
                     preferred_element_type=jnp.float32) * inv_l
        h = jnp.maximum(
            jnp.dot(c_ref[...], ew, preferred_element_type=jnp.float32)
            + b1_ref[...], 0.0)
        o = jnp.dot(h, w2_ref[...],
                    preferred_element_type=jnp.float32) + b2_ref[...]
        ol, orr = o[:, :P], o[:, P:]
        nl = jnp.maximum(
            jnp.sqrt(jnp.sum(ol * ol, axis=1, keepdims=True)), 1e-12)
        nr = jnp.maximum(
            jnp.sqrt(jnp.sum(orr * orr, axis=1, keepdims=True)), 1e-12)
        o_ref[...] = jnp.concatenate([ol / nl, orr / nr], axis=1)

    return pl.pallas_call(
        body,
        grid=(B2 // blk,),
        in_specs=[
            pl.BlockSpec((blk, V2), lambda i: (i, 0)),
            pl.BlockSpec((V2, E2), lambda i: (0, 0)),
            pl.BlockSpec((E2, H2), lambda i: (0, 0)),
            pl.BlockSpec((1, H2), lambda i: (0, 0)),
            pl.BlockSpec((H2, 2 * P), lambda i: (0, 0)),
            pl.BlockSpec((1, 2 * P), lambda i: (0, 0)),
        ],
        out_specs=pl.BlockSpec((blk, 2 * P), lambda i: (i, 0)),
        out_shape=jax.ShapeDtypeStruct((B2, 2 * P), jnp.float32),
    )(counts2, embd, w1d, b1d, w2d, b2d)


def _blockdiag(a, b):
    (m, n), (p, q) = a.shape, b.shape
    return jnp.concatenate([
        jnp.concatenate([a, jnp.zeros((m, q), a.dtype)], axis=1),
        jnp.concatenate([jnp.zeros((p, n), b.dtype), b], axis=1),
    ], axis=0)


def kernel(x, emb, W1, b1, W2, b2):
    B, L = x.shape
    V = emb.shape[0]
    H = W1.shape[1]
    P = W2.shape[1]
    assert B % (NW * NLANES) == 0
    counts2 = _sc_histogram(x.reshape(-1), B, L, V)
    embd = _blockdiag(emb, emb)
    w1d = _blockdiag(W1, W1)
    b1d = jnp.concatenate([b1, b1]).reshape(1, 2 * H)
    w2d = _blockdiag(W2, W2)
    b2d = jnp.concatenate([b2, b2]).reshape(1, 2 * P)
    out2 = _tc_mlp(counts2, embd, w1d, b1d, w2d, b2d, L, P)
    return out2.reshape(B, P)


# speedup vs baseline: 1.2539x; 1.0234x over previous
import functools

import jax
import jax.numpy as jnp
from jax import lax
from jax.experimental import pallas as pl
from jax.experimental.pallas import tpu as pltpu
from jax.experimental.pallas import tpu_sc as plsc

NUM_CORES = 2
NUM_SUBCORES = 16
NLANES = 16
NW = NUM_CORES * NUM_SUBCORES


def _sc_histogram(x_flat, B, L, V):
    rows_per_w = B // NW
    chunk_rows = 64
    n_chunks = rows_per_w // chunk_rows
    n_groups = chunk_rows // NLANES

    mesh = plsc.VectorSubcoreMesh(
        core_axis_name="c", subcore_axis_name="s",
        num_cores=NUM_CORES, num_subcores=NUM_SUBCORES)

    @functools.partial(
        pl.kernel,
        out_type=jax.ShapeDtypeStruct((B * V,), jnp.float32),
        mesh=mesh,
        compiler_params=pltpu.CompilerParams(
            needs_layout_passes=False, disable_bounds_checks=True),
        scratch_types=[
            pltpu.VMEM((chunk_rows * L,), jnp.int32),
            pltpu.VMEM((chunk_rows * L,), jnp.int32),
            pltpu.VMEM((rows_per_w * V,), jnp.float32),
            pltpu.SemaphoreType.DMA,
            pltpu.SemaphoreType.DMA,
        ],
    )
    def hist(x_hbm, out_hbm, xb0, xb1, counts, sem0, sem1):
        wid = lax.axis_index("s") * NUM_CORES + lax.axis_index("c")
        row0 = wid * rows_per_w

        lane = lax.iota(jnp.int32, NLANES)
        laneL = lane * L
        laneV = lane * V
        ones = jnp.full((NLANES,), 1.0, jnp.float32)
        zeros = jnp.zeros((NLANES,), jnp.float32)

        @plsc.parallel_loop(0, (rows_per_w * V) // NLANES, unroll=8)
        def _(j):
            counts[pl.ds(j * NLANES, NLANES)] = zeros

        xbufs = (xb0, xb1)
        sems = (sem0, sem1)

        def start_chunk(c):
            off = (row0 + c * chunk_rows) * L
            return pltpu.async_copy(
                x_hbm.at[pl.ds(off, chunk_rows * L)],
                xbufs[c % 2], sems[c % 2])

        pending = start_chunk(0)
        for c in range(n_chunks):
            pending.wait()
            if c + 1 < n_chunks:
                pending = start_chunk(c + 1)
            xb = xbufs[c % 2]
            for g in range(n_groups):
                src_base = laneL + (g * NLANES * L)
                dst_base = laneV + ((c * chunk_rows + g * NLANES) * V)

                @plsc.parallel_loop(0, L, unroll=8)
                def _(l):
                    v = plsc.load_gather(xb, [src_base + l])
                    plsc.addupdate_scatter(counts, [dst_base + v], ones)

        pltpu.sync_copy(counts,
                        out_hbm.at[pl.ds(row0 * V, rows_per_w * V)])

    return hist(x_flat)


def _tc_mlp(counts2, embd, w1d, b1d, w2d, b2d, L, P):
    B2, V2 = counts2.shape
    E2 = embd.shape[1]
    H2 = w1d.shape[1]
    blk = 2048
    inv_l = 1.0 / float(L)

    def body(c_ref, emb_ref, w1_ref, b1_ref, w2_ref, b2_ref, o_ref):
        ew = jnp.dot(emb_ref[...], w1_ref[...],
                     preferred_element_type=jnp.float32) * inv_l
        h = jnp.maximum(
            jnp.dot(c_ref[...], ew, preferred_element_type=jnp.float32)
            + b1_ref[...], 0.0)
        o = jnp.dot(h, w2_ref[...],
                    preferred_element_type=jnp.float32) + b2_ref[...]
        ol, orr = o[:, :P], o[:, P:]
        nl = jnp.maximum(
            jnp.sqrt(jnp.sum(ol * ol, axis=1, keepdims=True)), 1e-12)
        nr = jnp.maximum(
            jnp.sqrt(jnp.sum(orr * orr, axis=1, keepdims=True)), 1e-12)
        o_ref[...] = jnp.concatenate([ol / nl, orr / nr], axis=1)

    return pl.pallas_call(
        body,
        grid=(B2 // blk,),
        in_specs=[
            pl.BlockSpec((blk, V2), lambda i: (i, 0)),
            pl.BlockSpec((V2, E2), lambda i: (0, 0)),
            pl.BlockSpec((E2, H2), lambda i: (0, 0)),
            pl.BlockSpec((1, H2), lambda i: (0, 0)),
            pl.BlockSpec((H2, 2 * P), lambda i: (0, 0)),
            pl.BlockSpec((1, 2 * P), lambda i: (0, 0)),
        ],
        out_specs=pl.BlockSpec((blk, 2 * P), lambda i: (i, 0)),
        out_shape=jax.ShapeDtypeStruct((B2, 2 * P), jnp.float32),
    )(counts2, embd, w1d, b1d, w2d, b2d)


def _blockdiag(a, b):
    (m, n), (p, q) = a.shape, b.shape
    return jnp.concatenate([
        jnp.concatenate([a, jnp.zeros((m, q), a.dtype)], axis=1),
        jnp.concatenate([jnp.zeros((p, n), b.dtype), b], axis=1),
    ], axis=0)


def kernel(x, emb, W1, b1, W2, b2):
    B, L = x.shape
    V = emb.shape[0]
    H = W1.shape[1]
    P = W2.shape[1]
    assert B % (NW * NLANES) == 0
    counts2 = _sc_histogram(x.reshape(-1), B, L, V).reshape(B // 2, 2 * V)
    embd = _blockdiag(emb, emb)
    w1d = _blockdiag(W1, W1)
    b1d = jnp.concatenate([b1, b1]).reshape(1, 2 * H)
    w2d = _blockdiag(W2, W2)
    b2d = jnp.concatenate([b2, b2]).reshape(1, 2 * P)
    out2 = _tc_mlp(counts2, embd, w1d, b1d, w2d, b2d, L, P)
    return out2.reshape(B, P)


# speedup vs baseline: 1.3481x; 1.0751x over previous
import functools

import jax
import jax.numpy as jnp
from jax import lax
from jax.experimental import pallas as pl
from jax.experimental.pallas import tpu as pltpu
from jax.experimental.pallas import tpu_sc as plsc

NUM_CORES = 2
NUM_SUBCORES = 16
NLANES = 16
NW = NUM_CORES * NUM_SUBCORES


def _sc_histogram(x_flat, B, L, V):
    rows_per_w = B // NW
    chunk_rows = 64
    n_chunks = rows_per_w // chunk_rows
    n_groups = chunk_rows // NLANES

    mesh = plsc.VectorSubcoreMesh(
        core_axis_name="c", subcore_axis_name="s",
        num_cores=NUM_CORES, num_subcores=NUM_SUBCORES)

    @functools.partial(
        pl.kernel,
        out_type=jax.ShapeDtypeStruct((B * V,), jnp.float32),
        mesh=mesh,
        compiler_params=pltpu.CompilerParams(
            needs_layout_passes=False, disable_bounds_checks=True),
        scratch_types=[
            pltpu.VMEM((chunk_rows * L,), jnp.int32),
            pltpu.VMEM((chunk_rows * L,), jnp.int32),
            pltpu.VMEM((rows_per_w * V,), jnp.float32),
            pltpu.SemaphoreType.DMA,
            pltpu.SemaphoreType.DMA,
        ],
    )
    def hist(x_hbm, out_hbm, xb0, xb1, counts, sem0, sem1):
        wid = lax.axis_index("s") * NUM_CORES + lax.axis_index("c")
        row0 = wid * rows_per_w

        lane = lax.iota(jnp.int32, NLANES)
        laneL = lane * L
        laneV = lane * V
        ones = jnp.full((NLANES,), 1.0, jnp.float32)
        zeros = jnp.zeros((NLANES,), jnp.float32)

        @plsc.parallel_loop(0, (rows_per_w * V) // NLANES, unroll=8)
        def _(j):
            counts[pl.ds(j * NLANES, NLANES)] = zeros

        xbufs = (xb0, xb1)
        sems = (sem0, sem1)

        def start_chunk(c):
            off = (row0 + c * chunk_rows) * L
            return pltpu.async_copy(
                x_hbm.at[pl.ds(off, chunk_rows * L)],
                xbufs[c % 2], sems[c % 2])

        pending = start_chunk(0)
        for c in range(n_chunks):
            pending.wait()
            if c + 1 < n_chunks:
                pending = start_chunk(c + 1)
            xb = xbufs[c % 2]
            for g in range(n_groups):
                src_base = laneL + (g * NLANES * L)
                dst_base = laneV + ((c * chunk_rows + g * NLANES) * V)

                @plsc.parallel_loop(0, L, unroll=8)
                def _(l):
                    v = plsc.load_gather(xb, [src_base + l])
                    plsc.addupdate_scatter(counts, [dst_base + v], ones)

        pltpu.sync_copy(counts,
                        out_hbm.at[pl.ds(row0 * V, rows_per_w * V)])

    return hist(x_flat)


def _tc_mlp(counts, emb, W1, b1, W2, b2, L):
    B, V = counts.shape
    E = emb.shape[1]
    H = W1.shape[1]
    P = W2.shape[1]
    blk = 2048
    inv_l = 1.0 / float(L)

    def body(c_ref, emb_ref, w1_ref, b1_ref, w2_ref, b2_ref, o_ref):
        ew = jnp.dot(emb_ref[...], w1_ref[...],
                     preferred_element_type=jnp.float32) * inv_l
        h = jnp.maximum(
            jnp.dot(c_ref[...].astype(jnp.bfloat16),
                    ew.astype(jnp.bfloat16),
                    preferred_element_type=jnp.float32)
            + b1_ref[...], 0.0)
        o = jnp.dot(h.astype(jnp.bfloat16),
                    w2_ref[...].astype(jnp.bfloat16),
                    preferred_element_type=jnp.float32) + b2_ref[...]
        ss = jnp.sum(o * o, axis=1, keepdims=True)
        o_ref[...] = o / jnp.maximum(jnp.sqrt(ss), 1e-12)

    return pl.pallas_call(
        body,
        grid=(B // blk,),
        in_specs=[
            pl.BlockSpec((blk, V), lambda i: (i, 0)),
            pl.BlockSpec((V, E), lambda i: (0, 0)),
            pl.BlockSpec((E, H), lambda i: (0, 0)),
            pl.BlockSpec((1, H), lambda i: (0, 0)),
            pl.BlockSpec((H, P), lambda i: (0, 0)),
            pl.BlockSpec((1, P), lambda i: (0, 0)),
        ],
        out_specs=pl.BlockSpec((blk, P), lambda i: (i, 0)),
        out_shape=jax.ShapeDtypeStruct((B, P), jnp.float32),
    )(counts, emb, W1, b1.reshape(1, H), W2, b2.reshape(1, P))


def kernel(x, emb, W1, b1, W2, b2):
    B, L = x.shape
    V = emb.shape[0]
    assert B % (NW * NLANES) == 0
    counts = _sc_histogram(x.reshape(-1), B, L, V).reshape(B, V)
    return _tc_mlp(counts, emb, W1, b1, W2, b2, L)
